# Initial kernel scaffold; baseline (speedup 1.0000x reference)
import jax
import jax.numpy as jnp
from jax.experimental import pallas as pl


def kernel(x, W_q, W_k, W_v, W_o, sink, log_beta):
    raise NotImplementedError("write your pallas kernel here")


# speedup vs baseline: 9.4851x; 9.4851x over previous
import functools

import jax
import jax.numpy as jnp
from jax.experimental import pallas as pl
from jax.experimental.pallas import tpu as pltpu

_B, _T, _D = 1, 2048, 768
_H, _HD, _TK = 12, 64, 32
_NL = 128
_BT = 256
_NI = _T // _BT
_VW = 128


def _topk_threshold(x):
    n = _NL
    li = jax.lax.broadcasted_iota(jnp.int32, (1, n), 1)
    s = x
    k = 2
    while k <= n:
        j = k // 2
        while j >= 1:
            mask_j = (li & j) != 0
            p = jnp.where(mask_j, pltpu.roll(s, j, 1), pltpu.roll(s, n - j, 1))
            keep_min = ((li & k) == 0) == ((li & j) == 0)
            s = jnp.where(keep_min, jnp.minimum(s, p), jnp.maximum(s, p))
            j //= 2
        k *= 2
    return jax.lax.slice_in_dim(s, n - _TK, n - _TK + 1, axis=1)


def _proj_kernel(x_ref, wq_ref, wk_ref, wv_ref, qm_ref, km_ref, va_ref):
    x = x_ref[...]
    dims = (((1,), (1,)), ((), ()))
    q = jax.lax.dot_general(x, wq_ref[...], dims,
                            preferred_element_type=jnp.float32)
    q = jnp.maximum(q, 0.0)
    qm_ref[...] = jnp.where(q >= _topk_threshold(q), q, 0.0)
    kk = jax.lax.dot_general(x, wk_ref[...], dims,
                             preferred_element_type=jnp.float32)
    kk = jnp.maximum(kk, 0.0)
    km_ref[...] = jnp.where(kk >= _topk_threshold(kk), kk, 0.0)
    v = jax.lax.dot_general(x, wv_ref[...], dims,
                            preferred_element_type=jnp.float32)
    va_ref[...] = jnp.concatenate(
        [v, jnp.ones((_BT, 1), jnp.float32),
         jnp.zeros((_BT, _VW - _HD - 1), jnp.float32)], axis=1)


def _attn_kernel(lb_ref, qm_ref, km_ref, va_ref, sink_ref, wo_ref, out_ref):
    i = pl.program_id(0)
    h = pl.program_id(1)
    q = qm_ref[...]
    beta = jnp.exp(lb_ref[0, 0])
    r_iota = jax.lax.broadcasted_iota(jnp.int32, (_BT, _BT), 0)
    c_iota = jax.lax.broadcasted_iota(jnp.int32, (_BT, _BT), 1)
    rc = r_iota - c_iota

    def body(j, yd):
        kj = km_ref[pl.ds(j * _BT, _BT), :]
        s = jax.lax.dot_general(q, kj, (((1,), (1,)), ((), ())),
                                preferred_element_type=jnp.float32)
        keep = (rc + (i - j) * _BT) >= 0
        s = jnp.where(keep, s, 0.0)
        vj = va_ref[pl.ds(j * _BT, _BT), :]
        return yd + jax.lax.dot_general(
            s, vj, (((1,), (0,)), ((), ())),
            preferred_element_type=jnp.float32)

    yd = jax.lax.fori_loop(0, i + 1, body,
                           jnp.zeros((_BT, _VW), jnp.float32))
    y = yd[:, :_HD]
    denom = yd[:, _HD:_HD + 1]
    dws = denom + beta
    y = y / jnp.maximum(dws, 1e-12) + (beta / dws) * sink_ref[pl.ds(h, 1), :]
    o = jax.lax.dot_general(y, wo_ref[...], (((1,), (0,)), ((), ())),
                            preferred_element_type=jnp.float32)

    @pl.when(h == 0)
    def _():
        out_ref[...] = o

    @pl.when(h > 0)
    def _():
        out_ref[...] += o


@jax.jit
def _run(x2, W_q, W_k, W_v, W_o, sink, log_beta):
    qm, km, va = pl.pallas_call(
        _proj_kernel,
        grid=(_H, _NI),
        in_specs=[
            pl.BlockSpec((_BT, _D), lambda h, i: (i, 0)),
            pl.BlockSpec((_NL, _D), lambda h, i: (h, 0)),
            pl.BlockSpec((_NL, _D), lambda h, i: (h, 0)),
            pl.BlockSpec((_HD, _D), lambda h, i: (h, 0)),
        ],
        out_specs=[
            pl.BlockSpec((_BT, _NL), lambda h, i: (i, h)),
            pl.BlockSpec((_BT, _NL), lambda h, i: (i, h)),
            pl.BlockSpec((_BT, _VW), lambda h, i: (i, h)),
        ],
        out_shape=[
            jax.ShapeDtypeStruct((_T, _H * _NL), jnp.float32),
            jax.ShapeDtypeStruct((_T, _H * _NL), jnp.float32),
            jax.ShapeDtypeStruct((_T, _H * _VW), jnp.float32),
        ],
    )(x2, W_q, W_k, W_v)

    out = pl.pallas_call(
        _attn_kernel,
        grid=(_NI, _H),
        in_specs=[
            pl.BlockSpec((1, 1), lambda i, h: (0, 0), memory_space=pltpu.SMEM),
            pl.BlockSpec((_BT, _NL), lambda i, h: (i, h)),
            pl.BlockSpec((_T, _NL), lambda i, h: (0, h)),
            pl.BlockSpec((_T, _VW), lambda i, h: (0, h)),
            pl.BlockSpec((_H, _HD), lambda i, h: (0, 0)),
            pl.BlockSpec((_HD, _D), lambda i, h: (h, 0)),
        ],
        out_specs=pl.BlockSpec((_BT, _D), lambda i, h: (i, 0)),
        out_shape=jax.ShapeDtypeStruct((_T, _D), jnp.float32),
    )(log_beta.reshape(1, 1), qm, km, va, sink, W_o.T)
    return out


def kernel(x, W_q, W_k, W_v, W_o, sink, log_beta):
    out = _run(x.reshape(_T, _D), W_q, W_k, W_v, W_o, sink, log_beta)
    return out.reshape(_B, _T, _D)


# speedup vs baseline: 10.8069x; 1.1394x over previous
import functools

import jax
import jax.numpy as jnp
from jax.experimental import pallas as pl
from jax.experimental.pallas import tpu as pltpu

_B, _T, _D = 1, 2048, 768
_H, _HD, _TK = 12, 64, 32
_NL = 128
_BT = 256
_NI = _T // _BT
_VW = 128


def _topk_threshold(x):
    n = _NL
    li = jax.lax.broadcasted_iota(jnp.int32, (1, n), 1)
    s = x
    k = 2
    while k <= n:
        j = k // 2
        while j >= 1:
            mask_j = (li & j) != 0
            p = jnp.where(mask_j, pltpu.roll(s, j, 1), pltpu.roll(s, n - j, 1))
            keep_min = ((li & k) == 0) == ((li & j) == 0)
            s = jnp.where(keep_min, jnp.minimum(s, p), jnp.maximum(s, p))
            j //= 2
        k *= 2
    return jax.lax.slice_in_dim(s, n - _TK, n - _TK + 1, axis=1)


def _proj_kernel(x_ref, wq_ref, wk_ref, wv_ref, qm_ref, km_ref, va_ref):
    x = x_ref[...]
    dims = (((1,), (1,)), ((), ()))
    q = jax.lax.dot_general(x, wq_ref[...], dims,
                            preferred_element_type=jnp.float32)
    q = jnp.maximum(q, 0.0)
    kk = jax.lax.dot_general(x, wk_ref[...], dims,
                             preferred_element_type=jnp.float32)
    kk = jnp.maximum(kk, 0.0)
    t = _topk_threshold(jnp.concatenate([q, kk], axis=0))
    qm_ref[...] = jnp.where(q >= t[:_BT], q, 0.0).astype(jnp.bfloat16)
    km_ref[...] = jnp.where(kk >= t[_BT:], kk, 0.0).astype(jnp.bfloat16)
    v = jax.lax.dot_general(x, wv_ref[...], dims,
                            preferred_element_type=jnp.float32)
    va_ref[...] = jnp.concatenate(
        [v, jnp.ones((_BT, 1), jnp.float32),
         jnp.zeros((_BT, _VW - _HD - 1), jnp.float32)], axis=1).astype(jnp.bfloat16)


def _attn_kernel(lb_ref, qm_ref, km_ref, va_ref, sink_ref, wo_ref, out_ref):
    i = pl.program_id(0)
    h = pl.program_id(1)
    q = qm_ref[...]
    beta = jnp.exp(lb_ref[0, 0])
    r_iota = jax.lax.broadcasted_iota(jnp.int32, (_BT, _BT), 0)
    c_iota = jax.lax.broadcasted_iota(jnp.int32, (_BT, _BT), 1)
    rc = r_iota - c_iota

    def body(j, yd):
        kj = km_ref[pl.ds(j * _BT, _BT), :]
        s = jax.lax.dot_general(q, kj, (((1,), (1,)), ((), ())),
                                preferred_element_type=jnp.float32)
        keep = (rc + (i - j) * _BT) >= 0
        s = jnp.where(keep, s, 0.0).astype(jnp.bfloat16)
        vj = va_ref[pl.ds(j * _BT, _BT), :]
        return yd + jax.lax.dot_general(
            s, vj, (((1,), (0,)), ((), ())),
            preferred_element_type=jnp.float32)

    yd = jax.lax.fori_loop(0, i + 1, body,
                           jnp.zeros((_BT, _VW), jnp.float32))
    y = yd[:, :_HD]
    denom = yd[:, _HD:_HD + 1]
    dws = denom + beta
    y = y / jnp.maximum(dws, 1e-12) + (beta / dws) * sink_ref[pl.ds(h, 1), :]
    o = jax.lax.dot_general(y.astype(jnp.bfloat16), wo_ref[...],
                            (((1,), (0,)), ((), ())),
                            preferred_element_type=jnp.float32)

    @pl.when(h == 0)
    def _():
        out_ref[...] = o

    @pl.when(h > 0)
    def _():
        out_ref[...] += o


@jax.jit
def _run(x2, W_q, W_k, W_v, W_o, sink, log_beta):
    qm, km, va = pl.pallas_call(
        _proj_kernel,
        grid=(_H, _NI),
        in_specs=[
            pl.BlockSpec((_BT, _D), lambda h, i: (i, 0)),
            pl.BlockSpec((_NL, _D), lambda h, i: (h, 0)),
            pl.BlockSpec((_NL, _D), lambda h, i: (h, 0)),
            pl.BlockSpec((_HD, _D), lambda h, i: (h, 0)),
        ],
        out_specs=[
            pl.BlockSpec((_BT, _NL), lambda h, i: (i, h)),
            pl.BlockSpec((_BT, _NL), lambda h, i: (i, h)),
            pl.BlockSpec((_BT, _VW), lambda h, i: (i, h)),
        ],
        out_shape=[
            jax.ShapeDtypeStruct((_T, _H * _NL), jnp.bfloat16),
            jax.ShapeDtypeStruct((_T, _H * _NL), jnp.bfloat16),
            jax.ShapeDtypeStruct((_T, _H * _VW), jnp.bfloat16),
        ],
    )(x2, W_q, W_k, W_v)

    out = pl.pallas_call(
        _attn_kernel,
        grid=(_NI, _H),
        in_specs=[
            pl.BlockSpec((1, 1), lambda i, h: (0, 0), memory_space=pltpu.SMEM),
            pl.BlockSpec((_BT, _NL), lambda i, h: (i, h)),
            pl.BlockSpec((_T, _NL), lambda i, h: (0, h)),
            pl.BlockSpec((_T, _VW), lambda i, h: (0, h)),
            pl.BlockSpec((_H, _HD), lambda i, h: (0, 0)),
            pl.BlockSpec((_HD, _D), lambda i, h: (h, 0)),
        ],
        out_specs=pl.BlockSpec((_BT, _D), lambda i, h: (i, 0)),
        out_shape=jax.ShapeDtypeStruct((_T, _D), jnp.float32),
    )(log_beta.reshape(1, 1), qm, km, va, sink,
      W_o.T.astype(jnp.bfloat16))
    return out


def kernel(x, W_q, W_k, W_v, W_o, sink, log_beta):
    out = _run(x.reshape(_T, _D), W_q, W_k, W_v, W_o, sink, log_beta)
    return out.reshape(_B, _T, _D)


# speedup vs baseline: 12.3917x; 1.1466x over previous
import functools

import jax
import jax.numpy as jnp
from jax.experimental import pallas as pl
from jax.experimental.pallas import tpu as pltpu

_B, _T, _D = 1, 2048, 768
_H, _HD, _TK = 12, 64, 32
_NL = 128
_BT = 256
_NI = _T // _BT
_VW = 128


def _topk_threshold(x):
    n = _NL
    li = jax.lax.broadcasted_iota(jnp.int32, (1, n), 1)
    s = x
    k = 2
    while k <= n:
        j = k // 2
        while j >= 1:
            mask_j = (li & j) != 0
            p = jnp.where(mask_j, pltpu.roll(s, j, 1), pltpu.roll(s, n - j, 1))
            keep_min = ((li & k) == 0) == ((li & j) == 0)
            s = jnp.where(keep_min, jnp.minimum(s, p), jnp.maximum(s, p))
            j //= 2
        k *= 2
    return jax.lax.slice_in_dim(s, n - _TK, n - _TK + 1, axis=1)


def _proj_kernel(x_ref, wq_ref, wk_ref, wv_ref, qm_ref, km_ref, va_ref):
    x = x_ref[...]
    dims = (((1,), (1,)), ((), ()))
    q = jax.lax.dot_general(x, wq_ref[...], dims,
                            preferred_element_type=jnp.float32)
    q = jnp.maximum(q, 0.0)
    kk = jax.lax.dot_general(x, wk_ref[...], dims,
                             preferred_element_type=jnp.float32)
    kk = jnp.maximum(kk, 0.0)
    t = _topk_threshold(jnp.concatenate([q, kk], axis=0))
    qm_ref[...] = jnp.where(q >= t[:_BT], q, 0.0).astype(jnp.bfloat16)
    km_ref[...] = jnp.where(kk >= t[_BT:], kk, 0.0).astype(jnp.bfloat16)
    v = jax.lax.dot_general(x, wv_ref[...], dims,
                            preferred_element_type=jnp.float32)
    va_ref[...] = jnp.concatenate(
        [v, jnp.ones((_BT, 1), jnp.float32),
         jnp.zeros((_BT, _VW - _HD - 1), jnp.float32)], axis=1).astype(jnp.bfloat16)


def _attn_kernel(lb_ref, qm_ref, km_ref, va_ref, sink_ref, wo_ref, out_ref):
    i = pl.program_id(0)
    h = pl.program_id(1)
    q = qm_ref[...]
    beta = jnp.exp(lb_ref[0, 0])
    s = jax.lax.dot_general(q, km_ref[...], (((1,), (1,)), ((), ())),
                            preferred_element_type=jnp.float32)
    r_iota = jax.lax.broadcasted_iota(jnp.int32, (_BT, _T), 0)
    c_iota = jax.lax.broadcasted_iota(jnp.int32, (_BT, _T), 1)
    keep = (r_iota + i * _BT) >= c_iota
    s = jnp.where(keep, s, 0.0).astype(jnp.bfloat16)
    yd = jax.lax.dot_general(s, va_ref[...], (((1,), (0,)), ((), ())),
                             preferred_element_type=jnp.float32)
    y = yd[:, :_HD]
    denom = yd[:, _HD:_HD + 1]
    dws = denom + beta
    y = y / jnp.maximum(dws, 1e-12) + (beta / dws) * sink_ref[pl.ds(h, 1), :]
    o = jax.lax.dot_general(y.astype(jnp.bfloat16), wo_ref[...],
                            (((1,), (0,)), ((), ())),
                            preferred_element_type=jnp.float32)

    @pl.when(h == 0)
    def _():
        out_ref[...] = o

    @pl.when(h > 0)
    def _():
        out_ref[...] += o


@jax.jit
def _run(x2, W_q, W_k, W_v, W_o, sink, log_beta):
    qm, km, va = pl.pallas_call(
        _proj_kernel,
        grid=(_H, _NI),
        in_specs=[
            pl.BlockSpec((_BT, _D), lambda h, i: (i, 0)),
            pl.BlockSpec((_NL, _D), lambda h, i: (h, 0)),
            pl.BlockSpec((_NL, _D), lambda h, i: (h, 0)),
            pl.BlockSpec((_HD, _D), lambda h, i: (h, 0)),
        ],
        out_specs=[
            pl.BlockSpec((_BT, _NL), lambda h, i: (i, h)),
            pl.BlockSpec((_BT, _NL), lambda h, i: (i, h)),
            pl.BlockSpec((_BT, _VW), lambda h, i: (i, h)),
        ],
        out_shape=[
            jax.ShapeDtypeStruct((_T, _H * _NL), jnp.bfloat16),
            jax.ShapeDtypeStruct((_T, _H * _NL), jnp.bfloat16),
            jax.ShapeDtypeStruct((_T, _H * _VW), jnp.bfloat16),
        ],
    )(x2, W_q, W_k, W_v)

    out = pl.pallas_call(
        _attn_kernel,
        grid=(_NI, _H),
        in_specs=[
            pl.BlockSpec((1, 1), lambda i, h: (0, 0), memory_space=pltpu.SMEM),
            pl.BlockSpec((_BT, _NL), lambda i, h: (i, h)),
            pl.BlockSpec((_T, _NL), lambda i, h: (0, h)),
            pl.BlockSpec((_T, _VW), lambda i, h: (0, h)),
            pl.BlockSpec((_H, _HD), lambda i, h: (0, 0)),
            pl.BlockSpec((_HD, _D), lambda i, h: (h, 0)),
        ],
        out_specs=pl.BlockSpec((_BT, _D), lambda i, h: (i, 0)),
        out_shape=jax.ShapeDtypeStruct((_T, _D), jnp.float32),
    )(log_beta.reshape(1, 1), qm, km, va, sink,
      W_o.T.astype(jnp.bfloat16))
    return out


def kernel(x, W_q, W_k, W_v, W_o, sink, log_beta):
    out = _run(x.reshape(_T, _D), W_q, W_k, W_v, W_o, sink, log_beta)
    return out.reshape(_B, _T, _D)


# speedup vs baseline: 19.5627x; 1.5787x over previous
import functools

import numpy as np
import jax
import jax.numpy as jnp
from jax.experimental import pallas as pl
from jax.experimental.pallas import tpu as pltpu

_B, _T, _D = 1, 2048, 768
_H, _HD, _TK = 12, 64, 32
_NL = 128
_BT = 1024
_NI = _T // _BT
_VW = 128


def _topk_threshold(x):
    n = _NL
    lanes = np.arange(n)
    li = jax.lax.broadcasted_iota(jnp.int32, (1, n), 1)
    s = x
    k = 2
    while k <= n:
        j = k // 2
        while j >= 1:
            p = jnp.take_along_axis(s, jnp.broadcast_to(li ^ j, s.shape), axis=1)
            keep_min = ((li & k) == 0) == ((li & j) == 0)
            s = jnp.where(keep_min, jnp.minimum(s, p), jnp.maximum(s, p))
            j //= 2
        k *= 2
    return jax.lax.slice_in_dim(s, n - _TK, n - _TK + 1, axis=1)


def _proj_kernel(x_ref, wq_ref, wk_ref, wv_ref, qm_ref, km_ref, va_ref):
    x = x_ref[...]
    dims = (((1,), (1,)), ((), ()))
    q = jax.lax.dot_general(x, wq_ref[...], dims,
                            preferred_element_type=jnp.float32)
    q = jnp.maximum(q, 0.0)
    kk = jax.lax.dot_general(x, wk_ref[...], dims,
                             preferred_element_type=jnp.float32)
    kk = jnp.maximum(kk, 0.0)
    t = _topk_threshold(jnp.concatenate([q, kk], axis=0))
    qm_ref[...] = jnp.where(q >= t[:_BT], q, 0.0).astype(jnp.bfloat16)
    km_ref[...] = jnp.where(kk >= t[_BT:], kk, 0.0).astype(jnp.bfloat16)
    v = jax.lax.dot_general(x, wv_ref[...], dims,
                            preferred_element_type=jnp.float32)
    va_ref[...] = jnp.concatenate(
        [v, jnp.ones((_BT, 1), jnp.float32),
         jnp.zeros((_BT, _VW - _HD - 1), jnp.float32)], axis=1).astype(jnp.bfloat16)


def _attn_kernel(lb_ref, qm_ref, km_ref, va_ref, sink_ref, wo_ref, out_ref):
    i = pl.program_id(0)
    h = pl.program_id(1)
    q = qm_ref[...]
    beta = jnp.exp(lb_ref[0, 0])
    s = jax.lax.dot_general(q, km_ref[...], (((1,), (1,)), ((), ())),
                            preferred_element_type=jnp.float32)
    r_iota = jax.lax.broadcasted_iota(jnp.int32, (_BT, _T), 0)
    c_iota = jax.lax.broadcasted_iota(jnp.int32, (_BT, _T), 1)
    keep = (r_iota + i * _BT) >= c_iota
    s = jnp.where(keep, s, 0.0).astype(jnp.bfloat16)
    yd = jax.lax.dot_general(s, va_ref[...], (((1,), (0,)), ((), ())),
                             preferred_element_type=jnp.float32)
    y = yd[:, :_HD]
    denom = yd[:, _HD:_HD + 1]
    dws = denom + beta
    y = y / jnp.maximum(dws, 1e-12) + (beta / dws) * sink_ref[pl.ds(h, 1), :]
    o = jax.lax.dot_general(y.astype(jnp.bfloat16), wo_ref[...],
                            (((1,), (0,)), ((), ())),
                            preferred_element_type=jnp.float32)

    @pl.when(h == 0)
    def _():
        out_ref[...] = o

    @pl.when(h > 0)
    def _():
        out_ref[...] += o


@jax.jit
def _run(x2, W_q, W_k, W_v, W_o, sink, log_beta):
    qm, km, va = pl.pallas_call(
        _proj_kernel,
        grid=(_H, _NI),
        in_specs=[
            pl.BlockSpec((_BT, _D), lambda h, i: (i, 0)),
            pl.BlockSpec((_NL, _D), lambda h, i: (h, 0)),
            pl.BlockSpec((_NL, _D), lambda h, i: (h, 0)),
            pl.BlockSpec((_HD, _D), lambda h, i: (h, 0)),
        ],
        out_specs=[
            pl.BlockSpec((_BT, _NL), lambda h, i: (i, h)),
            pl.BlockSpec((_BT, _NL), lambda h, i: (i, h)),
            pl.BlockSpec((_BT, _VW), lambda h, i: (i, h)),
        ],
        out_shape=[
            jax.ShapeDtypeStruct((_T, _H * _NL), jnp.bfloat16),
            jax.ShapeDtypeStruct((_T, _H * _NL), jnp.bfloat16),
            jax.ShapeDtypeStruct((_T, _H * _VW), jnp.bfloat16),
        ],
    )(x2, W_q, W_k, W_v)

    out = pl.pallas_call(
        _attn_kernel,
        grid=(_NI, _H),
        in_specs=[
            pl.BlockSpec((1, 1), lambda i, h: (0, 0), memory_space=pltpu.SMEM),
            pl.BlockSpec((_BT, _NL), lambda i, h: (i, h)),
            pl.BlockSpec((_T, _NL), lambda i, h: (0, h)),
            pl.BlockSpec((_T, _VW), lambda i, h: (0, h)),
            pl.BlockSpec((_H, _HD), lambda i, h: (0, 0)),
            pl.BlockSpec((_HD, _D), lambda i, h: (h, 0)),
        ],
        out_specs=pl.BlockSpec((_BT, _D), lambda i, h: (i, 0)),
        out_shape=jax.ShapeDtypeStruct((_T, _D), jnp.float32),
    )(log_beta.reshape(1, 1), qm, km, va, sink,
      W_o.T.astype(jnp.bfloat16))
    return out


def kernel(x, W_q, W_k, W_v, W_o, sink, log_beta):
    out = _run(x.reshape(_T, _D), W_q, W_k, W_v, W_o, sink, log_beta)
    return out.reshape(_B, _T, _D)


# speedup vs baseline: 20.4275x; 1.0442x over previous
import functools

import numpy as np
import jax
import jax.numpy as jnp
from jax.experimental import pallas as pl
from jax.experimental.pallas import tpu as pltpu

_B, _T, _D = 1, 2048, 768
_H, _HD, _TK = 12, 64, 32
_NL = 128
_BT = 2048
_NI = _T // _BT
_VW = 128


def _topk_threshold(x):
    n = _NL
    lanes = np.arange(n)
    li = jax.lax.broadcasted_iota(jnp.int32, (1, n), 1)
    s = x
    k = 2
    while k <= n:
        j = k // 2
        while j >= 1:
            p = jnp.take_along_axis(s, jnp.broadcast_to(li ^ j, s.shape), axis=1)
            keep_min = ((li & k) == 0) == ((li & j) == 0)
            s = jnp.where(keep_min, jnp.minimum(s, p), jnp.maximum(s, p))
            j //= 2
        k *= 2
    return jax.lax.slice_in_dim(s, n - _TK, n - _TK + 1, axis=1)


def _proj_kernel(x_ref, wq_ref, wk_ref, wv_ref, qm_ref, km_ref, va_ref):
    x = x_ref[...]
    dims = (((1,), (1,)), ((), ()))
    q = jax.lax.dot_general(x, wq_ref[...], dims,
                            preferred_element_type=jnp.float32)
    q = jnp.maximum(q, 0.0)
    kk = jax.lax.dot_general(x, wk_ref[...], dims,
                             preferred_element_type=jnp.float32)
    kk = jnp.maximum(kk, 0.0)
    t = _topk_threshold(jnp.concatenate([q, kk], axis=0))
    qm_ref[...] = jnp.where(q >= t[:_BT], q, 0.0).astype(jnp.bfloat16)
    km_ref[...] = jnp.where(kk >= t[_BT:], kk, 0.0).astype(jnp.bfloat16)
    v = jax.lax.dot_general(x, wv_ref[...], dims,
                            preferred_element_type=jnp.float32)
    va_ref[...] = jnp.concatenate(
        [v, jnp.ones((_BT, 1), jnp.float32),
         jnp.zeros((_BT, _VW - _HD - 1), jnp.float32)], axis=1).astype(jnp.bfloat16)


def _attn_kernel(lb_ref, qm_ref, km_ref, va_ref, sink_ref, wo_ref, out_ref):
    i = pl.program_id(0)
    h = pl.program_id(1)
    q = qm_ref[...]
    beta = jnp.exp(lb_ref[0, 0])
    s = jax.lax.dot_general(q, km_ref[...], (((1,), (1,)), ((), ())),
                            preferred_element_type=jnp.float32)
    r_iota = jax.lax.broadcasted_iota(jnp.int32, (_BT, _T), 0)
    c_iota = jax.lax.broadcasted_iota(jnp.int32, (_BT, _T), 1)
    keep = (r_iota + i * _BT) >= c_iota
    s = jnp.where(keep, s, 0.0).astype(jnp.bfloat16)
    yd = jax.lax.dot_general(s, va_ref[...], (((1,), (0,)), ((), ())),
                             preferred_element_type=jnp.float32)
    y = yd[:, :_HD]
    denom = yd[:, _HD:_HD + 1]
    dws = denom + beta
    y = y / jnp.maximum(dws, 1e-12) + (beta / dws) * sink_ref[pl.ds(h, 1), :]
    o = jax.lax.dot_general(y.astype(jnp.bfloat16), wo_ref[...],
                            (((1,), (0,)), ((), ())),
                            preferred_element_type=jnp.float32)

    @pl.when(h == 0)
    def _():
        out_ref[...] = o

    @pl.when(h > 0)
    def _():
        out_ref[...] += o


@jax.jit
def _run(x2, W_q, W_k, W_v, W_o, sink, log_beta):
    qm, km, va = pl.pallas_call(
        _proj_kernel,
        grid=(_H, _NI),
        in_specs=[
            pl.BlockSpec((_BT, _D), lambda h, i: (i, 0)),
            pl.BlockSpec((_NL, _D), lambda h, i: (h, 0)),
            pl.BlockSpec((_NL, _D), lambda h, i: (h, 0)),
            pl.BlockSpec((_HD, _D), lambda h, i: (h, 0)),
        ],
        out_specs=[
            pl.BlockSpec((_BT, _NL), lambda h, i: (i, h)),
            pl.BlockSpec((_BT, _NL), lambda h, i: (i, h)),
            pl.BlockSpec((_BT, _VW), lambda h, i: (i, h)),
        ],
        out_shape=[
            jax.ShapeDtypeStruct((_T, _H * _NL), jnp.bfloat16),
            jax.ShapeDtypeStruct((_T, _H * _NL), jnp.bfloat16),
            jax.ShapeDtypeStruct((_T, _H * _VW), jnp.bfloat16),
        ],
    )(x2, W_q, W_k, W_v)

    out = pl.pallas_call(
        _attn_kernel,
        grid=(_NI, _H),
        in_specs=[
            pl.BlockSpec((1, 1), lambda i, h: (0, 0), memory_space=pltpu.SMEM),
            pl.BlockSpec((_BT, _NL), lambda i, h: (i, h)),
            pl.BlockSpec((_T, _NL), lambda i, h: (0, h)),
            pl.BlockSpec((_T, _VW), lambda i, h: (0, h)),
            pl.BlockSpec((_H, _HD), lambda i, h: (0, 0)),
            pl.BlockSpec((_HD, _D), lambda i, h: (h, 0)),
        ],
        out_specs=pl.BlockSpec((_BT, _D), lambda i, h: (i, 0)),
        out_shape=jax.ShapeDtypeStruct((_T, _D), jnp.float32),
    )(log_beta.reshape(1, 1), qm, km, va, sink,
      W_o.T.astype(jnp.bfloat16))
    return out


def kernel(x, W_q, W_k, W_v, W_o, sink, log_beta):
    out = _run(x.reshape(_T, _D), W_q, W_k, W_v, W_o, sink, log_beta)
    return out.reshape(_B, _T, _D)
